# baseline (speedup 1.0000x reference)
import functools

import jax
import jax.numpy as jnp
from jax import lax
from jax.experimental import pallas as pl
from jax.experimental.pallas import tpu as pltpu


def _rowmax_group(xg, iota, big):
    mg = jnp.max(xg, axis=1, keepdims=True)
    cand = jnp.where(xg == mg, iota, big)
    idx = jnp.min(cand, axis=1, keepdims=True)
    return idx, mg


def _rowmax_kernel(xa_ref, xb_ref, iota_ref, ida_ref, maxa_ref, idb_ref,
                   maxb_ref):
    R, n = xa_ref.shape
    RG = 8
    big = jnp.float32(n)
    iota = iota_ref[...]
    for x_ref, id_ref, max_ref in ((xa_ref, ida_ref, maxa_ref),
                                   (xb_ref, idb_ref, maxb_ref)):
        ids = []
        maxs = []
        for r0 in range(0, R, RG):
            idx, mg = _rowmax_group(x_ref[r0:r0 + RG, :], iota, big)
            ids.append(idx)
            maxs.append(mg)
        id_ref[...] = jnp.concatenate(ids, axis=0)
        max_ref[...] = jnp.concatenate(maxs, axis=0)


@functools.partial(jax.jit, static_argnames=("block_rows",))
def _rowmax(x2d, block_rows=64):
    rows, n = x2d.shape
    half = rows // 2
    grid = (half // block_rows,)
    iota8 = jnp.broadcast_to(
        jnp.arange(n, dtype=jnp.float32)[None, :], (8, n))
    xa = x2d[:half]
    xb = x2d[half:]
    ida, mxa, idb, mxb = pl.pallas_call(
        _rowmax_kernel,
        grid=grid,
        in_specs=[
            pl.BlockSpec((block_rows, n), lambda i: (i, 0)),
            pl.BlockSpec((block_rows, n), lambda i: (i, 0)),
            pl.BlockSpec((8, n), lambda i: (0, 0)),
        ],
        out_specs=[
            pl.BlockSpec((block_rows, 1), lambda i: (i, 0)),
            pl.BlockSpec((block_rows, 1), lambda i: (i, 0)),
            pl.BlockSpec((block_rows, 1), lambda i: (i, 0)),
            pl.BlockSpec((block_rows, 1), lambda i: (i, 0)),
        ],
        out_shape=[
            jax.ShapeDtypeStruct((half, 1), jnp.float32),
            jax.ShapeDtypeStruct((half, 1), jnp.float32),
            jax.ShapeDtypeStruct((half, 1), jnp.float32),
            jax.ShapeDtypeStruct((half, 1), jnp.float32),
        ],
        compiler_params=pltpu.CompilerParams(
            dimension_semantics=("arbitrary",),
        ),
    )(xa, xb, iota8)
    id_out = jnp.concatenate([ida, idb], axis=0)
    max_out = jnp.concatenate([mxa, mxb], axis=0)
    return id_out, max_out


def kernel(prob_vec):
    b, s, n = prob_vec.shape
    x2d = prob_vec.reshape(b * s, n)
    id_out, max_out = _rowmax(x2d)
    out = jnp.concatenate([id_out, max_out], axis=1)
    return out.reshape(b, s, 2)


# speedup vs baseline: 2.6891x; 2.6891x over previous
import functools

import jax
import jax.numpy as jnp
from jax import lax
from jax.experimental import pallas as pl
from jax.experimental.pallas import tpu as pltpu


def _rowmax_group(xg, iota, big):
    mg = jnp.max(xg, axis=1, keepdims=True)
    cand = jnp.where(xg == mg, iota, big)
    idx = jnp.min(cand, axis=1, keepdims=True)
    return idx, mg


def _rowmax_kernel(xa_ref, xb_ref, iota_ref, ida_ref, maxa_ref, idb_ref,
                   maxb_ref):
    R, n = xa_ref.shape
    RG = 8
    big = jnp.float32(n)
    iota = iota_ref[...]
    for x_ref, id_ref, max_ref in ((xa_ref, ida_ref, maxa_ref),
                                   (xb_ref, idb_ref, maxb_ref)):
        ids = []
        maxs = []
        for r0 in range(0, R, RG):
            idx, mg = _rowmax_group(x_ref[r0:r0 + RG, :], iota, big)
            ids.append(idx)
            maxs.append(mg)
        id_ref[...] = jnp.concatenate(ids, axis=0)
        max_ref[...] = jnp.concatenate(maxs, axis=0)


@functools.partial(jax.jit, static_argnames=("block_rows",))
def _rowmax(x2d, block_rows=64):
    rows, n = x2d.shape
    half = rows // 2
    grid = (half // block_rows,)
    iota8 = jnp.broadcast_to(
        jnp.arange(n, dtype=jnp.float32)[None, :], (8, n))
    nblk = half // block_rows
    ida, mxa, idb, mxb = pl.pallas_call(
        _rowmax_kernel,
        grid=grid,
        in_specs=[
            pl.BlockSpec((block_rows, n), lambda i: (i, 0)),
            pl.BlockSpec((block_rows, n), lambda i: (i + nblk, 0)),
            pl.BlockSpec((8, n), lambda i: (0, 0)),
        ],
        out_specs=[
            pl.BlockSpec((block_rows, 1), lambda i: (i, 0)),
            pl.BlockSpec((block_rows, 1), lambda i: (i, 0)),
            pl.BlockSpec((block_rows, 1), lambda i: (i, 0)),
            pl.BlockSpec((block_rows, 1), lambda i: (i, 0)),
        ],
        out_shape=[
            jax.ShapeDtypeStruct((half, 1), jnp.float32),
            jax.ShapeDtypeStruct((half, 1), jnp.float32),
            jax.ShapeDtypeStruct((half, 1), jnp.float32),
            jax.ShapeDtypeStruct((half, 1), jnp.float32),
        ],
        compiler_params=pltpu.CompilerParams(
            dimension_semantics=("arbitrary",),
        ),
    )(x2d, x2d, iota8)
    id_out = jnp.concatenate([ida, idb], axis=0)
    max_out = jnp.concatenate([mxa, mxb], axis=0)
    return id_out, max_out


def kernel(prob_vec):
    b, s, n = prob_vec.shape
    x2d = prob_vec.reshape(b * s, n)
    id_out, max_out = _rowmax(x2d)
    out = jnp.concatenate([id_out, max_out], axis=1)
    return out.reshape(b, s, 2)
